# baseline (speedup 1.0000x reference)
import jax
import jax.numpy as jnp
from jax.experimental import pallas as pl

_B, _A, _NC, _DFL = 32, 8400, 80, 16
_TA = 2800
_NBLK = _A // _TA


def _loss_kernel(pd_ref, pb_ref, tb_ref, ts_ref, ap_ref, m_ref, tss_ref,
                 box_ref, dfl_ref):
    bi = pl.program_id(0)
    ai = pl.program_id(1)

    @pl.when(jnp.logical_and(bi == 0, ai == 0))
    def _init():
        box_ref[:, :] = jnp.zeros_like(box_ref)
        dfl_ref[:, :] = jnp.zeros_like(dfl_ref)

    pb = pb_ref[0]
    tb = tb_ref[0]
    ts = ts_ref[0]
    ap = ap_ref[...]
    m = m_ref[0]

    w = jnp.sum(ts, axis=-1, keepdims=True) * m

    ix1 = jnp.maximum(pb[:, 0:1], tb[:, 0:1])
    iy1 = jnp.maximum(pb[:, 1:2], tb[:, 1:2])
    ix2 = jnp.minimum(pb[:, 2:3], tb[:, 2:3])
    iy2 = jnp.minimum(pb[:, 3:4], tb[:, 3:4])
    inter = jnp.maximum(ix2 - ix1, 0.0) * jnp.maximum(iy2 - iy1, 0.0)
    area1 = (pb[:, 2:3] - pb[:, 0:1]) * (pb[:, 3:4] - pb[:, 1:2])
    area2 = (tb[:, 2:3] - tb[:, 0:1]) * (tb[:, 3:4] - tb[:, 1:2])
    iou = inter / (area1 + area2 - inter + 1e-7)
    box_part = jnp.sum((1.0 - iou) * w)

    pd = pd_ref[0]
    tgt = jnp.concatenate([ap - tb[:, 0:2], tb[:, 2:4] - ap], axis=-1)
    tgt = jnp.clip(tgt, 0.0, _DFL - 0.01)
    lane = jax.lax.broadcasted_iota(jnp.int32, (_TA, _DFL + 1), 1)
    dfl_acc = jnp.zeros((_TA, 1), jnp.float32)
    for j in range(4):
        t = tgt[:, j:j + 1]
        tl = jnp.floor(t)
        wl = tl + 1.0 - t
        wr = 1.0 - wl
        logits = pd[:, j * (_DFL + 1):(j + 1) * (_DFL + 1)]
        mx = jnp.max(logits, axis=-1, keepdims=True)
        sh = logits - mx
        lse = jnp.log(jnp.sum(jnp.exp(sh), axis=-1, keepdims=True))
        tli = tl.astype(jnp.int32)
        sel_l = jnp.sum(jnp.where(lane == tli, sh, 0.0), -1, keepdims=True)
        sel_r = jnp.sum(jnp.where(lane == tli + 1, sh, 0.0), -1, keepdims=True)
        dfl_acc = dfl_acc + (lse - sel_l) * wl + (lse - sel_r) * wr
    dfl_part = jnp.sum(dfl_acc * 0.25 * w)

    box_ref[:, :] += jnp.reshape(box_part, (1, 1))
    dfl_ref[:, :] += jnp.reshape(dfl_part, (1, 1))

    @pl.when(jnp.logical_and(bi == _B - 1, ai == _NBLK - 1))
    def _finalize():
        inv = 1.0 / tss_ref[0, 0]
        box_ref[:, :] *= inv
        dfl_ref[:, :] *= inv


def kernel(pred_dist, pred_bboxes, anchor_points, target_bboxes,
           target_scores, target_scores_sum, fg_mask):
    mask = fg_mask.astype(jnp.float32).reshape(_B, _A, 1)
    tss = target_scores_sum.reshape(1, 1)
    out = pl.pallas_call(
        _loss_kernel,
        grid=(_B, _NBLK),
        in_specs=[
            pl.BlockSpec((1, _TA, 4 * (_DFL + 1)), lambda b, a: (b, a, 0)),
            pl.BlockSpec((1, _TA, 4), lambda b, a: (b, a, 0)),
            pl.BlockSpec((1, _TA, 4), lambda b, a: (b, a, 0)),
            pl.BlockSpec((1, _TA, _NC), lambda b, a: (b, a, 0)),
            pl.BlockSpec((_TA, 2), lambda b, a: (a, 0)),
            pl.BlockSpec((1, _TA, 1), lambda b, a: (b, a, 0)),
            pl.BlockSpec((1, 1), lambda b, a: (0, 0)),
        ],
        out_specs=[
            pl.BlockSpec((1, 1), lambda b, a: (0, 0)),
            pl.BlockSpec((1, 1), lambda b, a: (0, 0)),
        ],
        out_shape=[jax.ShapeDtypeStruct((1, 1), jnp.float32),
                   jax.ShapeDtypeStruct((1, 1), jnp.float32)],
    )(pred_dist, pred_bboxes, target_bboxes, target_scores, anchor_points,
      mask, tss)
    return (out[0][0, 0], out[1][0, 0])


# speedup vs baseline: 5.2221x; 5.2221x over previous
import jax
import jax.numpy as jnp
from jax.experimental import pallas as pl
from jax.experimental.pallas import tpu as pltpu

_B, _A, _NC, _DFL = 32, 8400, 80, 16
_NCH = 4 * (_DFL + 1)


def _loss_kernel(pd_ref, ts_ref, tb_ref, ap_ref, pbt_ref, tbt_ref, m_ref,
                 tss_ref, box_ref, dfl_ref, accb_ref, accd_ref):
    b = pl.program_id(0)
    f32 = jnp.float32

    @pl.when(b == 0)
    def _init():
        accb_ref[...] = jnp.zeros_like(accb_ref)
        accd_ref[...] = jnp.zeros_like(accd_ref)

    P = pd_ref[0]
    TS = ts_ref[0]
    TB = tb_ref[0]
    AP = ap_ref[...]
    pbt = pbt_ref[0]
    tbt = tbt_ref[0]
    mrow = m_ref[0]

    lane_contract = (((1,), (1,)), ((), ()))

    w = jax.lax.dot_general(jnp.ones((1, _NC), f32), TS, lane_contract,
                            preferred_element_type=f32)
    w = w * mrow

    ix = jnp.minimum(pbt[2:3], tbt[2:3]) - jnp.maximum(pbt[0:1], tbt[0:1])
    iy = jnp.minimum(pbt[3:4], tbt[3:4]) - jnp.maximum(pbt[1:2], tbt[1:2])
    inter = jnp.maximum(ix, 0.0) * jnp.maximum(iy, 0.0)
    area1 = (pbt[2:3] - pbt[0:1]) * (pbt[3:4] - pbt[1:2])
    area2 = (tbt[2:3] - tbt[0:1]) * (tbt[3:4] - tbt[1:2])
    iou = inter / (area1 + area2 - inter + 1e-7)
    accb_ref[...] += (1.0 - iou) * w

    lane68 = jax.lax.broadcasted_iota(jnp.int32, (1, _NCH), 1)
    g1 = lane68 // (_DFL + 1)
    sgn = jnp.where(g1 >= 2, 1.0, -1.0).astype(f32)
    r4 = jax.lax.broadcasted_iota(jnp.int32, (4, _NCH), 0)
    g4 = jax.lax.broadcasted_iota(jnp.int32, (4, _NCH), 1) // (_DFL + 1)
    G4 = jnp.where(r4 == g4, 1.0, 0.0).astype(f32)
    M1 = G4 * sgn
    r2 = jax.lax.broadcasted_iota(jnp.int32, (2, _NCH), 0)
    g2 = jax.lax.broadcasted_iota(jnp.int32, (2, _NCH), 1) // (_DFL + 1)
    M2 = jnp.where(r2 == (g2 % 2), 1.0, 0.0).astype(f32) * (-sgn)

    row_contract = (((1,), (0,)), ((), ()))
    t68 = (jax.lax.dot_general(TB, M1, row_contract, preferred_element_type=f32)
           + jax.lax.dot_general(AP, M2, row_contract, preferred_element_type=f32))
    t68 = jnp.clip(t68, 0.0, _DFL - 0.01)

    l17 = (lane68 % (_DFL + 1)).astype(f32)
    coef = jnp.maximum(1.0 - jnp.abs(t68 - l17), 0.0)

    E = jnp.exp(jnp.clip(P, -85.0, 85.0))
    S = jax.lax.dot_general(G4, E, lane_contract,
                            preferred_element_type=f32)
    lse = jnp.sum(jnp.log(S), axis=0, keepdims=True)

    sel = jax.lax.dot_general(jnp.ones((1, _NCH), f32), P * coef,
                              lane_contract,
                              preferred_element_type=f32)
    accd_ref[...] += (lse - sel) * 0.25 * w

    @pl.when(b == _B - 1)
    def _finalize():
        inv = 1.0 / tss_ref[0, 0]
        box_ref[...] = jnp.reshape(jnp.sum(accb_ref[...]) * inv, (1, 1))
        dfl_ref[...] = jnp.reshape(jnp.sum(accd_ref[...]) * inv, (1, 1))


def kernel(pred_dist, pred_bboxes, anchor_points, target_bboxes,
           target_scores, target_scores_sum, fg_mask):
    f32 = jnp.float32
    pbt = jnp.swapaxes(pred_bboxes, 1, 2)
    tbt = jnp.swapaxes(target_bboxes, 1, 2)
    mask = fg_mask.astype(f32).reshape(_B, 1, _A)
    tss = target_scores_sum.reshape(1, 1)
    out = pl.pallas_call(
        _loss_kernel,
        grid=(_B,),
        in_specs=[
            pl.BlockSpec((1, _A, _NCH), lambda b: (b, 0, 0)),
            pl.BlockSpec((1, _A, _NC), lambda b: (b, 0, 0)),
            pl.BlockSpec((1, _A, 4), lambda b: (b, 0, 0)),
            pl.BlockSpec((_A, 2), lambda b: (0, 0)),
            pl.BlockSpec((1, 4, _A), lambda b: (b, 0, 0)),
            pl.BlockSpec((1, 4, _A), lambda b: (b, 0, 0)),
            pl.BlockSpec((1, 1, _A), lambda b: (b, 0, 0)),
            pl.BlockSpec((1, 1), lambda b: (0, 0)),
        ],
        out_specs=[
            pl.BlockSpec((1, 1), lambda b: (0, 0)),
            pl.BlockSpec((1, 1), lambda b: (0, 0)),
        ],
        out_shape=[jax.ShapeDtypeStruct((1, 1), f32),
                   jax.ShapeDtypeStruct((1, 1), f32)],
        scratch_shapes=[pltpu.VMEM((1, _A), f32),
                        pltpu.VMEM((1, _A), f32)],
    )(pred_dist, target_scores, target_bboxes, anchor_points, pbt, tbt,
      mask, tss)
    return (out[0][0, 0], out[1][0, 0])
